# Initial kernel scaffold; baseline (speedup 1.0000x reference)
import jax
import jax.numpy as jnp
from jax.experimental import pallas as pl


def kernel(x, edge_index, W1, att_src1, att_dst1, b1, W2_src, W2_dst, att_src2, att_dst2, b2):
    raise NotImplementedError("write your pallas kernel here")


# speedup vs baseline: 1.1483x; 1.1483x over previous
import jax
import jax.numpy as jnp
from jax.experimental import pallas as pl

N_NODES = 10000
D_HID = 250


def _mm_body(x_ref, w_ref, o_ref):
    o_ref[...] = jnp.dot(x_ref[...], w_ref[...],
                         preferred_element_type=jnp.float32)


def _matmul(x, w):
    return pl.pallas_call(
        _mm_body,
        out_shape=jax.ShapeDtypeStruct((x.shape[0], w.shape[1]), jnp.float32),
    )(x, w)


def _ext_weights(W_src, W_dst, att_src, att_dst):
    d_in = W_src.shape[0]
    return jnp.concatenate(
        [W_src, (W_src @ att_src)[:, None], (W_dst @ att_dst)[:, None],
         jnp.zeros((d_in, 4), jnp.float32)], axis=1)


def _edge_phase(h, a_s, a_d, src, dst, b):
    e = jax.nn.leaky_relu(a_s[src] + a_d[dst], negative_slope=0.2)
    m = jax.ops.segment_max(e, dst, num_segments=N_NODES)
    ex = jnp.exp(e - m[dst])
    denom = jax.ops.segment_sum(ex, dst, num_segments=N_NODES)
    alpha = ex / denom[dst]
    msg = alpha[:, None] * h[src]
    out = jax.ops.segment_sum(msg, dst, num_segments=N_NODES)
    return out + b


def kernel(x, edge_index, W1, att_src1, att_dst1, b1,
           W2_src, W2_dst, att_src2, att_dst2, b2):
    loops = jnp.arange(N_NODES, dtype=jnp.int32)
    ei = jnp.concatenate(
        [edge_index.astype(jnp.int32),
         jnp.stack([loops, loops], axis=0)], axis=1)
    src, dst = ei[0], ei[1]

    H1 = _matmul(x, _ext_weights(W1, W1, att_src1, att_dst1))
    h1, a_s1, a_d1 = H1[:, :D_HID], H1[:, D_HID], H1[:, D_HID + 1]
    x2 = jax.nn.relu(_edge_phase(h1, a_s1, a_d1, src, dst, b1))

    H2 = _matmul(x2, _ext_weights(W2_src, W2_dst, att_src2, att_dst2))
    h2, a_s2, a_d2 = H2[:, :D_HID], H2[:, D_HID], H2[:, D_HID + 1]
    return _edge_phase(h2, a_s2, a_d2, src, dst, b2)


# speedup vs baseline: 8.8283x; 7.6882x over previous
import functools

import jax
import jax.numpy as jnp
from jax import lax
from jax.experimental import pallas as pl
from jax.experimental.pallas import tpu as pltpu
from jax.experimental.pallas import tpu_sc as plsc

N = 10000
NP = 10240
DH = 250
NSUB = 16
EB = 128
NB = 162
EPAD = NSUB * NB * EB
HW = 128
NR = 5120
ACC = NR + EB
F32 = jnp.float32


def _ext_weights(W_src, W_dst, att_src, att_dst):
    d_in = W_src.shape[0]
    return jnp.concatenate(
        [W_src, (W_src @ att_src)[:, None], (W_dst @ att_dst)[:, None],
         jnp.zeros((d_in, 4), F32)], axis=1)


def _split_h(H, h0, h1, as_ref, ad_ref):
    for hi, h in enumerate((h0, h1)):
        h[...] = jnp.zeros((NP, HW), F32)
        h[0:N, 0:HW] = H[:, hi * HW:(hi + 1) * HW]
    as_ref[...] = jnp.zeros((NP,), F32)
    ad_ref[...] = jnp.zeros((NP,), F32)
    as_ref[0:N] = H[:, 250]
    ad_ref[0:N] = H[:, 251]


def _l1_body(x_ref, w_ref, h0, h1, as_ref, ad_ref):
    H = jnp.dot(x_ref[...], w_ref[...], preferred_element_type=F32)
    _split_h(H, h0, h1, as_ref, ad_ref)


def _normed(a0, a1, den_ref, b_ref):
    d = den_ref[...][0:N]
    h = jnp.concatenate(
        [a0[...][0:N, 0:HW], a1[...][0:N, 0:DH - HW]], axis=1)
    return h / d[:, None] + b_ref[...][None, :]


def _mid_body(a0, a1, den_ref, b_ref, w_ref, h0, h1, as_ref, ad_ref):
    x2 = jnp.maximum(_normed(a0, a1, den_ref, b_ref), 0.0)
    H = jnp.dot(x2, w_ref[...], preferred_element_type=F32)
    _split_h(H, h0, h1, as_ref, ad_ref)


def _out_body(a0, a1, den_ref, b_ref, o_ref):
    o_ref[...] = _normed(a0, a1, den_ref, b_ref)


_node_tables = [jax.ShapeDtypeStruct((NP, HW), F32)] * 2 + [
    jax.ShapeDtypeStruct((NP,), F32),
    jax.ShapeDtypeStruct((NP,), F32)]


def _sc_agg(src3, dst3, a_s, a_d, h0, h1):
    mesh = plsc.VectorSubcoreMesh(core_axis_name="c", subcore_axis_name="s")

    @functools.partial(
        pl.kernel, mesh=mesh,
        out_type=[jax.ShapeDtypeStruct((NP, HW), F32)] * 2 + [
            jax.ShapeDtypeStruct((NP,), F32)],
        scratch_types=[
            pltpu.VMEM((NB, EB), jnp.int32),
            pltpu.VMEM((NB, EB), jnp.int32),
            pltpu.VMEM((NB, EB), F32),
            pltpu.VMEM((EB,), F32),
            pltpu.VMEM((EB,), F32),
            pltpu.VMEM((EB,), jnp.int32),
            pltpu.VMEM((EB, HW), F32),
            pltpu.VMEM_SHARED((ACC, HW), F32),
            pltpu.VMEM_SHARED((NP,), F32),
            pltpu.SemaphoreType.DMA,
        ])
    def k(src_h, dst_h, as_h, ad_h, h0_h, h1_h,
          o0_h, o1_h, den_h,
          srcv, dstv, exv, ga, gb, adj, rows, sacc, sden, sem):
        c = lax.axis_index("c")
        s = lax.axis_index("s")

        def zero_rows():
            def zrow(r, carry):
                for k16 in range(HW // 16):
                    rows[r, pl.ds(k16 * 16, 16)] = jnp.zeros((16,), F32)
                return carry
            lax.fori_loop(0, EB, zrow, 0)

        def zero_stripe():
            for kk in range(5):
                pltpu.sync_copy(rows.at[pl.ds(0, 64)],
                                sacc.at[pl.ds(s * 320 + kk * 64, 64)])

        zero_rows()

        @pl.when(jnp.logical_and(c == 0, s == 0))
        def _():
            def zden(j, carry):
                pltpu.sync_copy(rows.at[0], sden.at[pl.ds(j * HW, HW)])
                return carry
            lax.fori_loop(0, NP // HW, zden, 0)

        plsc.subcore_barrier()

        pltpu.sync_copy(src_h.at[s], srcv)
        pltpu.sync_copy(dst_h.at[s], dstv)

        def p1(j, carry):
            pltpu.async_copy(as_h.at[srcv.at[j]], ga, sem).wait()
            pltpu.async_copy(ad_h.at[dstv.at[j]], gb, sem).wait()
            for k8 in range(8):
                sl = pl.ds(k8 * 16, 16)
                e = ga[sl] + gb[sl]
                e = jnp.where(e >= 0.0, e, e * 0.2)
                exv[j, sl] = jnp.exp(e)
            return carry
        lax.fori_loop(0, NB, p1, 0)

        @pl.when(c == 0)
        def _():
            def p1b(j, carry):
                pltpu.sync_copy(exv.at[j], sden.at[dstv.at[j]], add=True)
                return carry
            lax.fori_loop(0, NB, p1b, 0)

        def p2_pass(base):
            def p2(j, carry):
                @pl.when(c == 0)
                def _():
                    pltpu.async_copy(h0_h.at[srcv.at[j]], rows, sem).wait()

                @pl.when(c == 1)
                def _():
                    pltpu.async_copy(h1_h.at[srcv.at[j]], rows, sem).wait()
                for g in range(8):
                    sl = pl.ds(g * 16, 16)
                    dv = dstv[j, sl] - base
                    ok = jnp.logical_and(dv >= 0, dv < NR)
                    adj[sl] = jnp.where(ok, dv, NR)
                    wv = exv[j, sl]
                    for i in range(16):
                        w = wv[i]
                        r = g * 16 + i
                        for k16 in range(HW // 16):
                            cs = pl.ds(k16 * 16, 16)
                            rows[r, cs] = rows[r, cs] * w
                pltpu.sync_copy(rows, sacc.at[adj], add=True)
                return carry
            lax.fori_loop(0, NB, p2, 0)

        def drain(base):
            @pl.when(c == 0)
            def _():
                pltpu.sync_copy(sacc.at[pl.ds(s * 320, 320)],
                                o0_h.at[pl.ds(base + s * 320, 320)])

            @pl.when(c == 1)
            def _():
                pltpu.sync_copy(sacc.at[pl.ds(s * 320, 320)],
                                o1_h.at[pl.ds(base + s * 320, 320)])

        for p in range(2):
            if p == 1:
                zero_rows()
            zero_stripe()
            plsc.subcore_barrier()
            p2_pass(p * NR)
            plsc.subcore_barrier()
            drain(p * NR)

        plsc.subcore_barrier()

        @pl.when(jnp.logical_and(c == 0, s == 0))
        def _():
            def wbd(j, carry):
                pltpu.sync_copy(sden.at[pl.ds(j * 2048, 2048)],
                                den_h.at[pl.ds(j * 2048, 2048)])
                return carry
            lax.fori_loop(0, 5, wbd, 0)

    return k(src3, dst3, a_s, a_d, h0, h1)


def kernel(x, edge_index, W1, att_src1, att_dst1, b1,
           W2_src, W2_dst, att_src2, att_dst2, b2):
    ei = edge_index.astype(jnp.int32)
    loops = jnp.arange(N, dtype=jnp.int32)
    pad = jnp.full((EPAD - (ei.shape[1] + N),), N, jnp.int32)
    src3 = jnp.concatenate([ei[0], loops, pad]).reshape(NSUB, NB, EB)
    dst3 = jnp.concatenate([ei[1], loops, pad]).reshape(NSUB, NB, EB)

    q = pl.pallas_call(_l1_body, out_shape=_node_tables)(
        x, _ext_weights(W1, W1, att_src1, att_dst1))
    agg1 = _sc_agg(src3, dst3, q[2], q[3], q[0], q[1])

    q = pl.pallas_call(_mid_body, out_shape=_node_tables)(
        *agg1, b1, _ext_weights(W2_src, W2_dst, att_src2, att_dst2))
    agg2 = _sc_agg(src3, dst3, q[2], q[3], q[0], q[1])

    return pl.pallas_call(
        _out_body, out_shape=jax.ShapeDtypeStruct((N, DH), F32))(
            *agg2, b2)
